# Initial kernel scaffold; baseline (speedup 1.0000x reference)
import jax
import jax.numpy as jnp
from jax.experimental import pallas as pl


def kernel(points):
    raise NotImplementedError("write your pallas kernel here")


# speedup vs baseline: 5.5611x; 5.5611x over previous
import functools

import jax
import jax.numpy as jnp
from jax import lax
from jax.experimental import pallas as pl
from jax.experimental.pallas import tpu as pltpu
from jax.experimental.pallas import tpu_sc as plsc

B = 32
N = 16384
NP = 64
PS = 32
L = 16
NCHUNK = N // L
SHIFT = 22
NB = 512
CAP = 1024
IMAX = 2**31 - 1


def _splat_i(s):
    return jnp.full((L,), s, dtype=jnp.int32)


def _splat_f(s):
    return jnp.full((L,), s, dtype=jnp.float32)


def _rbf16(v):
    u = plsc.bitcast(v, jnp.int32)
    r = (u + 32767 + (lax.shift_right_logical(u, 16) & 1)) & jnp.int32(-65536)
    return plsc.bitcast(r, jnp.float32)


def _sc_body(pts, patches_o, centers_o, xv, yv, zv, x2v, dv, ukv, histv,
             candk, candi, cbuf, pbuf):
    lanes = lax.iota(jnp.int32, L)
    lane0 = lanes == 0
    wid = lax.axis_index("s") * 2 + lax.axis_index("c")
    base = wid * (3 * N)

    pltpu.sync_copy(pts.at[pl.ds(base, N)], xv)
    pltpu.sync_copy(pts.at[pl.ds(base + N, N)], yv)
    pltpu.sync_copy(pts.at[pl.ds(base + 2 * N, N)], zv)

    def x2_body(i, _):
        for u in range(8):
            s = pl.ds((i * 8 + u) * L, L)
            x, y, z = xv[s], yv[s], zv[s]
            x2v[s] = (x * x + y * y) + z * z
            dv[s] = _splat_f(jnp.inf)
        return 0

    lax.fori_loop(0, NCHUNK // 8, x2_body, 0)

    def fps_step(t, cur):
        ci = _splat_i(cur)
        px = plsc.load_gather(xv, [ci])
        py = plsc.load_gather(yv, [ci])
        pz = plsc.load_gather(zv, [ci])
        plsc.store_scatter(cbuf, [_splat_i(t * 3)], px, mask=lane0)
        plsc.store_scatter(cbuf, [_splat_i(t * 3 + 1)], py, mask=lane0)
        plsc.store_scatter(cbuf, [_splat_i(t * 3 + 2)], pz, mask=lane0)

        def chunk(i, carry):
            m, idx = carry
            for u in range(8):
                c = i * 8 + u
                s = pl.ds(c * L, L)
                dx = xv[s] - px
                dy = yv[s] - py
                dz = zv[s] - pz
                d = (dx * dx + dy * dy) + dz * dz
                nd = jnp.minimum(dv[s], d)
                dv[s] = nd
                upd = nd > m
                m = jnp.where(upd, nd, m)
                idx = jnp.where(upd, _splat_i(c * L) + lanes, idx)
            return m, idx

        m, idx = lax.fori_loop(0, NCHUNK // 8, chunk,
                               (_splat_f(-jnp.inf), _splat_i(0)))
        gm = jnp.max(m)
        return jnp.min(jnp.where(m == gm, idx, IMAX))

    lax.fori_loop(0, NP, fps_step, jnp.int32(0))

    def center_step(p, _):
        cx = plsc.load_gather(cbuf, [_splat_i(p * 3)])
        cy = plsc.load_gather(cbuf, [_splat_i(p * 3 + 1)])
        cz = plsc.load_gather(cbuf, [_splat_i(p * 3 + 2)])
        c2 = (cx * cx + cy * cy) + cz * cz
        cxb, cyb, czb = _rbf16(cx), _rbf16(cy), _rbf16(cz)

        def hzero(i, _):
            histv[pl.ds(i * L, L)] = _splat_i(0)
            return 0

        lax.fori_loop(0, NB * L // L, hzero, 0)

        one = _splat_i(1)
        zf = _splat_f(0.0)
        nbm1 = _splat_i(NB - 1)

        def pass1(i, mn):
            for u in range(8):
                c = i * 8 + u
                s = pl.ds(c * L, L)
                xb = _rbf16(xv[s])
                yb = _rbf16(yv[s])
                zb = _rbf16(zv[s])
                dot = (cxb * xb + cyb * yb) + czb * zb
                d2 = (c2 + x2v[s]) - (dot + dot)
                ub = plsc.bitcast(d2, jnp.int32)
                ukv[s] = ub ^ lax.shift_right_logical(
                    lax.shift_right_arithmetic(ub, 31), 1)
                bkt = jnp.minimum(
                    lax.shift_right_arithmetic(
                        plsc.bitcast(jnp.maximum(d2, zf), jnp.int32),
                        SHIFT), nbm1)
                plsc.addupdate_scatter(
                    histv, [lax.shift_left(bkt, 4) + lanes], one)
                mn = jnp.minimum(mn, d2)
            return mn

        mn = lax.fori_loop(0, NCHUNK // 8, pass1, _splat_f(jnp.inf))
        b0 = jnp.min(jnp.minimum(
            lax.shift_right_arithmetic(
                plsc.bitcast(jnp.maximum(mn, zf), jnp.int32), SHIFT),
            nbm1))

        def tcond(carry):
            bkt, cum = carry
            return jnp.logical_and(cum < PS, bkt < NB)

        def tbody(carry):
            bkt, cum = carry
            row = histv[pl.ds(bkt * L, L)]
            return bkt + 1, cum + jnp.sum(row)

        tend, _ = lax.while_loop(tcond, tbody, (b0, jnp.int32(0)))
        thr = _splat_i(lax.shift_left(tend, SHIFT))

        def collect(i, off):
            for u in range(8):
                c = i * 8 + u
                s = pl.ds(c * L, L)
                uk = ukv[s]
                msk = uk < thr
                pos = plsc.cumsum(jnp.where(msk, 1, 0)) - 1
                addr = jnp.minimum(off + pos, _splat_i(CAP + 15))
                plsc.store_scatter(candk, [addr], uk, mask=msk)
                plsc.store_scatter(candi, [addr],
                                   _splat_i(c * L) + lanes, mask=msk)
                off = off + plsc.all_reduce_population_count(msk)
            return off

        off = lax.fori_loop(0, NCHUNK // 8, collect, _splat_i(0))
        cnum = jnp.minimum(jnp.max(off), jnp.int32(CAP))
        candk[pl.ds(cnum, L)] = _splat_i(IMAX)
        nv = lax.shift_right_arithmetic(cnum + 15, 4)

        def sel_step(t, carry):
            gk, gi = carry
            gkv, giv = _splat_i(gk), _splat_i(gi)

            def scan(v, c2_):
                bk, bi = c2_
                s = pl.ds(v * L, L)
                k = candk[s]
                i2 = candi[s]
                hit = jnp.logical_and(k == gkv, i2 == giv)
                k = jnp.where(hit, IMAX, k)
                candk[s] = k
                lt = jnp.logical_or(
                    k < bk, jnp.logical_and(k == bk, i2 < bi))
                return jnp.where(lt, k, bk), jnp.where(lt, i2, bi)

            bk, bi = lax.fori_loop(0, nv, scan, (_splat_i(IMAX),
                                                 _splat_i(IMAX)))
            ngk = jnp.min(bk)
            ngi = jnp.min(jnp.where(bk == ngk, bi, IMAX))
            giv = _splat_i(jnp.minimum(ngi, jnp.int32(N - 1)))
            pa = _splat_i(p * (PS * 3) + t * 3)
            plsc.store_scatter(pbuf, [pa],
                               plsc.load_gather(xv, [giv]) - cx, mask=lane0)
            plsc.store_scatter(pbuf, [pa + 1],
                               plsc.load_gather(yv, [giv]) - cy, mask=lane0)
            plsc.store_scatter(pbuf, [pa + 2],
                               plsc.load_gather(zv, [giv]) - cz, mask=lane0)
            return ngk, ngi

        lax.fori_loop(0, PS, sel_step, (jnp.int32(IMAX), jnp.int32(-1)))
        return 0

    lax.fori_loop(0, NP, center_step, 0)

    pltpu.sync_copy(pbuf, patches_o.at[pl.ds(wid * (NP * PS * 3),
                                             NP * PS * 3)])
    pltpu.sync_copy(cbuf, centers_o.at[pl.ds(wid * (NP * 3), NP * 3)])


@jax.jit
def kernel(points):
    pts = jnp.transpose(points, (0, 2, 1)).reshape(-1)
    f32 = jnp.float32
    run = pl.kernel(
        _sc_body,
        out_type=(
            jax.ShapeDtypeStruct((B * NP * PS * 3,), f32),
            jax.ShapeDtypeStruct((B * NP * 3,), f32),
        ),
        mesh=plsc.VectorSubcoreMesh(core_axis_name="c", subcore_axis_name="s"),
        compiler_params=pltpu.CompilerParams(
            use_tc_tiling_on_sc=False, needs_layout_passes=False),
        scratch_types=(
            pltpu.VMEM((N,), f32),
            pltpu.VMEM((N,), f32),
            pltpu.VMEM((N,), f32),
            pltpu.VMEM((N,), f32),
            pltpu.VMEM((N,), f32),
            pltpu.VMEM((N,), jnp.int32),
            pltpu.VMEM((NB * L,), jnp.int32),
            pltpu.VMEM((CAP + 32,), jnp.int32),
            pltpu.VMEM((CAP + 32,), jnp.int32),
            pltpu.VMEM((NP * 3,), f32),
            pltpu.VMEM((NP * PS * 3,), f32),
        ),
    )
    pf, cf = run(pts)
    return (pf.reshape(B, NP, PS, 3), cf.reshape(B, NP, 3))


# speedup vs baseline: 5.9960x; 1.0782x over previous
import functools

import jax
import jax.numpy as jnp
from jax import lax
from jax.experimental import pallas as pl
from jax.experimental.pallas import tpu as pltpu
from jax.experimental.pallas import tpu_sc as plsc

B = 32
N = 16384
NP = 64
PS = 32
L = 16
NCHUNK = N // L
SHIFT = 22
NB = 512
CAP = 1024
IMAX = 2**31 - 1


def _splat_i(s):
    return jnp.full((L,), s, dtype=jnp.int32)


def _splat_f(s):
    return jnp.full((L,), s, dtype=jnp.float32)


def _rbf16(v):
    u = plsc.bitcast(v, jnp.int32)
    r = (u + 32767 + (lax.shift_right_logical(u, 16) & 1)) & jnp.int32(-65536)
    return plsc.bitcast(r, jnp.float32)


def _sc_body(pts, patches_o, centers_o, xv, yv, zv, x2v, dv, ukv, histv,
             candk, candi, cbuf, pbuf):
    lanes = lax.iota(jnp.int32, L)
    lane0 = lanes == 0
    wid = lax.axis_index("s") * 2 + lax.axis_index("c")
    base = wid * (3 * N)

    pltpu.sync_copy(pts.at[pl.ds(base, N)], xv)
    pltpu.sync_copy(pts.at[pl.ds(base + N, N)], yv)
    pltpu.sync_copy(pts.at[pl.ds(base + 2 * N, N)], zv)

    def x2_body(i, _):
        for u in range(8):
            s = pl.ds((i * 8 + u) * L, L)
            x, y, z = xv[s], yv[s], zv[s]
            x2v[s] = (x * x + y * y) + z * z
            dv[s] = _splat_f(jnp.inf)
        return 0

    lax.fori_loop(0, NCHUNK // 8, x2_body, 0)

    def fps_step(t, cur):
        ci = _splat_i(cur)
        px = plsc.load_gather(xv, [ci])
        py = plsc.load_gather(yv, [ci])
        pz = plsc.load_gather(zv, [ci])
        plsc.store_scatter(cbuf, [_splat_i(t * 3)], px, mask=lane0)
        plsc.store_scatter(cbuf, [_splat_i(t * 3 + 1)], py, mask=lane0)
        plsc.store_scatter(cbuf, [_splat_i(t * 3 + 2)], pz, mask=lane0)

        def chunk(i, carry):
            ms, idxs = list(carry[0]), list(carry[1])
            for u in range(8):
                k = u % 4
                c = i * 8 + u
                s = pl.ds(c * L, L)
                dx = xv[s] - px
                dy = yv[s] - py
                dz = zv[s] - pz
                d = (dx * dx + dy * dy) + dz * dz
                nd = jnp.minimum(dv[s], d)
                dv[s] = nd
                upd = nd > ms[k]
                ms[k] = jnp.where(upd, nd, ms[k])
                idxs[k] = jnp.where(upd, _splat_i(c * L) + lanes, idxs[k])
            return tuple(ms), tuple(idxs)

        ninf = _splat_f(-jnp.inf)
        zi = _splat_i(0)
        ms, idxs = lax.fori_loop(0, NCHUNK // 8, chunk,
                                 ((ninf,) * 4, (zi,) * 4))
        m, idx = ms[0], idxs[0]
        for k in range(1, 4):
            better = jnp.logical_or(
                ms[k] > m, jnp.logical_and(ms[k] == m, idxs[k] < idx))
            m = jnp.where(better, ms[k], m)
            idx = jnp.where(better, idxs[k], idx)
        gm = jnp.max(m)
        return jnp.min(jnp.where(m == gm, idx, IMAX))

    lax.fori_loop(0, NP, fps_step, jnp.int32(0))

    def pack_body(i, _):
        for u in range(8):
            s = pl.ds((i * 8 + u) * L, L)
            ux = plsc.bitcast(_rbf16(xv[s]), jnp.int32)
            uy = plsc.bitcast(_rbf16(yv[s]), jnp.int32)
            dv[s] = plsc.bitcast(
                ux | lax.shift_right_logical(uy, 16), jnp.float32)
        return 0

    lax.fori_loop(0, NCHUNK // 8, pack_body, 0)

    def center_step(p, _):
        cx = plsc.load_gather(cbuf, [_splat_i(p * 3)])
        cy = plsc.load_gather(cbuf, [_splat_i(p * 3 + 1)])
        cz = plsc.load_gather(cbuf, [_splat_i(p * 3 + 2)])
        c2 = (cx * cx + cy * cy) + cz * cz
        cxb, cyb, czb = _rbf16(cx), _rbf16(cy), _rbf16(cz)

        def hzero(i, _):
            for u in range(8):
                histv[pl.ds((i * 8 + u) * L, L)] = _splat_i(0)
            return 0

        lax.fori_loop(0, NB // 8, hzero, 0)

        one = _splat_i(1)
        zf = _splat_f(0.0)
        nbm1 = _splat_i(NB - 1)

        def pass1(i, mn):
            for u in range(8):
                c = i * 8 + u
                s = pl.ds(c * L, L)
                pk = plsc.bitcast(dv[s], jnp.int32)
                xb = plsc.bitcast(pk & jnp.int32(-65536), jnp.float32)
                yb = plsc.bitcast(lax.shift_left(pk, 16), jnp.float32)
                zb = _rbf16(zv[s])
                dot = (cxb * xb + cyb * yb) + czb * zb
                d2 = (c2 + x2v[s]) - (dot + dot)
                ub = plsc.bitcast(d2, jnp.int32)
                ukv[s] = ub ^ lax.shift_right_logical(
                    lax.shift_right_arithmetic(ub, 31), 1)
                bkt = jnp.minimum(
                    lax.shift_right_arithmetic(
                        plsc.bitcast(jnp.maximum(d2, zf), jnp.int32),
                        SHIFT), nbm1)
                plsc.addupdate_scatter(
                    histv, [lax.shift_left(bkt, 4) + lanes], one)
                mn = jnp.minimum(mn, d2)
            return mn

        mn = lax.fori_loop(0, NCHUNK // 8, pass1, _splat_f(jnp.inf))
        b0 = jnp.min(jnp.minimum(
            lax.shift_right_arithmetic(
                plsc.bitcast(jnp.maximum(mn, zf), jnp.int32), SHIFT),
            nbm1))

        def tcond(carry):
            bkt, cum = carry
            return jnp.logical_and(cum < PS, bkt < NB)

        def tbody(carry):
            bkt, cum = carry
            row = histv[pl.ds(bkt * L, L)]
            return bkt + 1, cum + jnp.sum(row)

        tend, _ = lax.while_loop(tcond, tbody, (b0, jnp.int32(0)))
        thr = _splat_i(lax.shift_left(tend, SHIFT))

        def collect(i, off):
            for u in range(8):
                c = i * 8 + u
                s = pl.ds(c * L, L)
                uk = ukv[s]
                msk = uk < thr
                pos = plsc.cumsum(jnp.where(msk, 1, 0)) - 1
                addr = jnp.minimum(off + pos, _splat_i(CAP + 15))
                plsc.store_scatter(candk, [addr], uk, mask=msk)
                plsc.store_scatter(candi, [addr],
                                   _splat_i(c * L) + lanes, mask=msk)
                off = off + plsc.all_reduce_population_count(msk)
            return off

        off = lax.fori_loop(0, NCHUNK // 8, collect, _splat_i(0))
        cnum = jnp.minimum(jnp.max(off), jnp.int32(CAP))
        candk[pl.ds(cnum, L)] = _splat_i(IMAX)
        nv = lax.shift_right_arithmetic(cnum + 15, 4)

        def sel_step(t, carry):
            gk, gi = carry
            gkv, giv = _splat_i(gk), _splat_i(gi)

            def scan(v, c2_):
                bk, bi = c2_
                s = pl.ds(v * L, L)
                k = candk[s]
                i2 = candi[s]
                hit = jnp.logical_and(k == gkv, i2 == giv)
                k = jnp.where(hit, IMAX, k)
                candk[s] = k
                lt = jnp.logical_or(
                    k < bk, jnp.logical_and(k == bk, i2 < bi))
                return jnp.where(lt, k, bk), jnp.where(lt, i2, bi)

            bk, bi = lax.fori_loop(0, nv, scan, (_splat_i(IMAX),
                                                 _splat_i(IMAX)))
            ngk = jnp.min(bk)
            ngi = jnp.min(jnp.where(bk == ngk, bi, IMAX))
            giv = _splat_i(jnp.minimum(ngi, jnp.int32(N - 1)))
            pa = _splat_i(p * (PS * 3) + t * 3)
            plsc.store_scatter(pbuf, [pa],
                               plsc.load_gather(xv, [giv]) - cx, mask=lane0)
            plsc.store_scatter(pbuf, [pa + 1],
                               plsc.load_gather(yv, [giv]) - cy, mask=lane0)
            plsc.store_scatter(pbuf, [pa + 2],
                               plsc.load_gather(zv, [giv]) - cz, mask=lane0)
            return ngk, ngi

        lax.fori_loop(0, PS, sel_step, (jnp.int32(IMAX), jnp.int32(-1)))
        return 0

    lax.fori_loop(0, NP, center_step, 0)

    pltpu.sync_copy(pbuf, patches_o.at[pl.ds(wid * (NP * PS * 3),
                                             NP * PS * 3)])
    pltpu.sync_copy(cbuf, centers_o.at[pl.ds(wid * (NP * 3), NP * 3)])


@jax.jit
def kernel(points):
    pts = jnp.transpose(points, (0, 2, 1)).reshape(-1)
    f32 = jnp.float32
    run = pl.kernel(
        _sc_body,
        out_type=(
            jax.ShapeDtypeStruct((B * NP * PS * 3,), f32),
            jax.ShapeDtypeStruct((B * NP * 3,), f32),
        ),
        mesh=plsc.VectorSubcoreMesh(core_axis_name="c", subcore_axis_name="s"),
        compiler_params=pltpu.CompilerParams(
            use_tc_tiling_on_sc=False, needs_layout_passes=False),
        scratch_types=(
            pltpu.VMEM((N,), f32),
            pltpu.VMEM((N,), f32),
            pltpu.VMEM((N,), f32),
            pltpu.VMEM((N,), f32),
            pltpu.VMEM((N,), f32),
            pltpu.VMEM((N,), jnp.int32),
            pltpu.VMEM((NB * L,), jnp.int32),
            pltpu.VMEM((CAP + 32,), jnp.int32),
            pltpu.VMEM((CAP + 32,), jnp.int32),
            pltpu.VMEM((NP * 3,), f32),
            pltpu.VMEM((NP * PS * 3,), f32),
        ),
    )
    pf, cf = run(pts)
    return (pf.reshape(B, NP, PS, 3), cf.reshape(B, NP, 3))


# speedup vs baseline: 14.2758x; 2.3809x over previous
import functools

import jax
import jax.numpy as jnp
from jax import lax
from jax.experimental import pallas as pl
from jax.experimental.pallas import tpu as pltpu
from jax.experimental.pallas import tpu_sc as plsc

B = 32
N = 16384
NP = 64
PS = 32
L = 16
NCHUNK = N // L
SHIFT = 22
NB = 512
CAP = 1024
IMAX = 2**31 - 1


def _splat_i(s):
    return jnp.full((L,), s, dtype=jnp.int32)


def _splat_f(s):
    return jnp.full((L,), s, dtype=jnp.float32)


def _rbf16(v):
    u = plsc.bitcast(v, jnp.int32)
    r = (u + 32767 + (lax.shift_right_logical(u, 16) & 1)) & jnp.int32(-65536)
    return plsc.bitcast(r, jnp.float32)


def _sc_body(pts, patches_o, centers_o, xv, yv, zv, x2v, dv, ukv, histv,
             candk, candi, cbuf, pbuf):
    lanes = lax.iota(jnp.int32, L)
    lane0 = lanes == 0
    wid = lax.axis_index("s") * 2 + lax.axis_index("c")
    base = wid * (3 * N)

    pltpu.sync_copy(pts.at[pl.ds(base, N)], xv)
    pltpu.sync_copy(pts.at[pl.ds(base + N, N)], yv)
    pltpu.sync_copy(pts.at[pl.ds(base + 2 * N, N)], zv)

    def x2_body(i, _):
        for u in range(8):
            s = pl.ds((i * 8 + u) * L, L)
            x, y, z = xv[s], yv[s], zv[s]
            x2v[s] = (x * x + y * y) + z * z
            dv[s] = _splat_f(jnp.inf)
        return 0

    lax.fori_loop(0, NCHUNK // 8, x2_body, 0)

    def fps_step(t, cur):
        ci = _splat_i(cur)
        px = plsc.load_gather(xv, [ci])
        py = plsc.load_gather(yv, [ci])
        pz = plsc.load_gather(zv, [ci])
        plsc.store_scatter(cbuf, [_splat_i(t * 3)], px, mask=lane0)
        plsc.store_scatter(cbuf, [_splat_i(t * 3 + 1)], py, mask=lane0)
        plsc.store_scatter(cbuf, [_splat_i(t * 3 + 2)], pz, mask=lane0)

        def chunk(i, carry):
            ms, idxs = list(carry[0]), list(carry[1])
            U = 8
            ss = [pl.ds((i * 8 + u) * L, L) for u in range(U)]
            dx = [xv[s] - px for s in ss]
            dy = [yv[s] - py for s in ss]
            dz = [zv[s] - pz for s in ss]
            od = [dv[s] for s in ss]
            sq = [(dx[u] * dx[u] + dy[u] * dy[u]) for u in range(U)]
            d = [sq[u] + dz[u] * dz[u] for u in range(U)]
            nd = [jnp.minimum(od[u], d[u]) for u in range(U)]
            for u in range(U):
                dv[ss[u]] = nd[u]
            upd = [nd[u] > ms[u % 4] for u in range(4)]
            for u in range(4):
                ms[u] = jnp.where(upd[u], nd[u], ms[u])
                idxs[u] = jnp.where(upd[u], _splat_i((i * 8 + u) * L) + lanes,
                                    idxs[u])
            upd2 = [nd[4 + u] > ms[u] for u in range(4)]
            for u in range(4):
                ms[u] = jnp.where(upd2[u], nd[4 + u], ms[u])
                idxs[u] = jnp.where(upd2[u],
                                    _splat_i((i * 8 + 4 + u) * L) + lanes,
                                    idxs[u])
            return tuple(ms), tuple(idxs)

        ninf = _splat_f(-jnp.inf)
        zi = _splat_i(0)
        ms, idxs = lax.fori_loop(0, NCHUNK // 8, chunk,
                                 ((ninf,) * 4, (zi,) * 4))
        m, idx = ms[0], idxs[0]
        for k in range(1, 4):
            better = jnp.logical_or(
                ms[k] > m, jnp.logical_and(ms[k] == m, idxs[k] < idx))
            m = jnp.where(better, ms[k], m)
            idx = jnp.where(better, idxs[k], idx)
        gm = jnp.max(m)
        return jnp.min(jnp.where(m == gm, idx, IMAX))

    lax.fori_loop(0, NP, fps_step, jnp.int32(0))

    def pack_body(i, _):
        for u in range(8):
            s = pl.ds((i * 8 + u) * L, L)
            ux = plsc.bitcast(_rbf16(xv[s]), jnp.int32)
            uy = plsc.bitcast(_rbf16(yv[s]), jnp.int32)
            dv[s] = plsc.bitcast(
                ux | lax.shift_right_logical(uy, 16), jnp.float32)
        return 0

    lax.fori_loop(0, NCHUNK // 8, pack_body, 0)

    def center_step(p, _):
        cx = plsc.load_gather(cbuf, [_splat_i(p * 3)])
        cy = plsc.load_gather(cbuf, [_splat_i(p * 3 + 1)])
        cz = plsc.load_gather(cbuf, [_splat_i(p * 3 + 2)])
        c2 = (cx * cx + cy * cy) + cz * cz
        cxb, cyb, czb = _rbf16(cx), _rbf16(cy), _rbf16(cz)

        def hzero(i, _):
            for u in range(8):
                histv[pl.ds((i * 8 + u) * L, L)] = _splat_i(0)
            return 0

        lax.fori_loop(0, NB // 8, hzero, 0)

        one = _splat_i(1)
        zf = _splat_f(0.0)
        nbm1 = _splat_i(NB - 1)

        def pass1(i, mn):
            U = 8
            ss = [pl.ds((i * 8 + u) * L, L) for u in range(U)]
            pk = [plsc.bitcast(dv[s], jnp.int32) for s in ss]
            zr = [zv[s] for s in ss]
            xx = [x2v[s] for s in ss]
            xb = [plsc.bitcast(p_ & jnp.int32(-65536), jnp.float32)
                  for p_ in pk]
            yb = [plsc.bitcast(lax.shift_left(p_, 16), jnp.float32)
                  for p_ in pk]
            zb = [_rbf16(z) for z in zr]
            dt = [cxb * xb[u] + cyb * yb[u] for u in range(U)]
            dot = [dt[u] + czb * zb[u] for u in range(U)]
            d2 = [(c2 + xx[u]) - (dot[u] + dot[u]) for u in range(U)]
            ub = [plsc.bitcast(v, jnp.int32) for v in d2]
            uk = [ub[u] ^ lax.shift_right_logical(
                lax.shift_right_arithmetic(ub[u], 31), 1) for u in range(U)]
            for u in range(U):
                ukv[ss[u]] = uk[u]
            bkt = [jnp.minimum(
                lax.shift_right_arithmetic(
                    plsc.bitcast(jnp.maximum(d2[u], zf), jnp.int32),
                    SHIFT), nbm1) for u in range(U)]
            for u in range(U):
                plsc.addupdate_scatter(
                    histv, [lax.shift_left(bkt[u], 4) + lanes], one)
            m1 = [jnp.minimum(d2[2 * u], d2[2 * u + 1]) for u in range(4)]
            m2 = [jnp.minimum(m1[2 * u], m1[2 * u + 1]) for u in range(2)]
            return jnp.minimum(mn, jnp.minimum(m2[0], m2[1]))

        mn = lax.fori_loop(0, NCHUNK // 8, pass1, _splat_f(jnp.inf))
        b0 = jnp.min(jnp.minimum(
            lax.shift_right_arithmetic(
                plsc.bitcast(jnp.maximum(mn, zf), jnp.int32), SHIFT),
            nbm1))

        def tcond(carry):
            bkt, cum = carry
            return jnp.logical_and(cum < PS, bkt < NB)

        def tbody(carry):
            bkt, cum = carry
            row = histv[pl.ds(bkt * L, L)]
            return bkt + 1, cum + jnp.sum(row)

        tend, _ = lax.while_loop(tcond, tbody, (b0, jnp.int32(0)))
        thr = _splat_i(lax.shift_left(tend, SHIFT))

        def collect(i, off):
            U = 8
            ss = [pl.ds((i * 8 + u) * L, L) for u in range(U)]
            uks = [ukv[s] for s in ss]
            msks = [uk < thr for uk in uks]
            csum = [plsc.cumsum(jnp.where(m_, 1, 0)) for m_ in msks]
            pcs = [plsc.all_reduce_population_count(m_) for m_ in msks]
            offs = [off]
            for u in range(U):
                offs.append(offs[u] + pcs[u])
            capv = _splat_i(CAP + 15)
            addr = [jnp.minimum(offs[u] + (csum[u] - 1), capv)
                    for u in range(U)]
            for u in range(U):
                plsc.store_scatter(candk, [addr[u]], uks[u], mask=msks[u])
                plsc.store_scatter(candi, [addr[u]],
                                   _splat_i((i * 8 + u) * L) + lanes,
                                   mask=msks[u])
            return offs[U]

        off = lax.fori_loop(0, NCHUNK // 8, collect, _splat_i(0))
        cnum = jnp.minimum(jnp.max(off), jnp.int32(CAP))
        candk[pl.ds(cnum, L)] = _splat_i(IMAX)
        nv = lax.shift_right_arithmetic(cnum + 15, 4)

        def sel_step(t, carry):
            gk, gi = carry
            gkv, giv = _splat_i(gk), _splat_i(gi)

            def scan(v, c2_):
                bk, bi = c2_
                s = pl.ds(v * L, L)
                k = candk[s]
                i2 = candi[s]
                hit = jnp.logical_and(k == gkv, i2 == giv)
                k = jnp.where(hit, IMAX, k)
                candk[s] = k
                lt = jnp.logical_or(
                    k < bk, jnp.logical_and(k == bk, i2 < bi))
                return jnp.where(lt, k, bk), jnp.where(lt, i2, bi)

            bk, bi = lax.fori_loop(0, nv, scan, (_splat_i(IMAX),
                                                 _splat_i(IMAX)))
            ngk = jnp.min(bk)
            ngi = jnp.min(jnp.where(bk == ngk, bi, IMAX))
            giv = _splat_i(jnp.minimum(ngi, jnp.int32(N - 1)))
            pa = _splat_i(p * (PS * 3) + t * 3)
            plsc.store_scatter(pbuf, [pa],
                               plsc.load_gather(xv, [giv]) - cx, mask=lane0)
            plsc.store_scatter(pbuf, [pa + 1],
                               plsc.load_gather(yv, [giv]) - cy, mask=lane0)
            plsc.store_scatter(pbuf, [pa + 2],
                               plsc.load_gather(zv, [giv]) - cz, mask=lane0)
            return ngk, ngi

        lax.fori_loop(0, PS, sel_step, (jnp.int32(IMAX), jnp.int32(-1)))
        return 0

    lax.fori_loop(0, NP, center_step, 0)

    pltpu.sync_copy(pbuf, patches_o.at[pl.ds(wid * (NP * PS * 3),
                                             NP * PS * 3)])
    pltpu.sync_copy(cbuf, centers_o.at[pl.ds(wid * (NP * 3), NP * 3)])


@jax.jit
def kernel(points):
    pts = jnp.transpose(points, (0, 2, 1)).reshape(-1)
    f32 = jnp.float32
    run = pl.kernel(
        _sc_body,
        out_type=(
            jax.ShapeDtypeStruct((B * NP * PS * 3,), f32),
            jax.ShapeDtypeStruct((B * NP * 3,), f32),
        ),
        mesh=plsc.VectorSubcoreMesh(core_axis_name="c", subcore_axis_name="s"),
        compiler_params=pltpu.CompilerParams(
            use_tc_tiling_on_sc=False, needs_layout_passes=False),
        scratch_types=(
            pltpu.VMEM((N,), f32),
            pltpu.VMEM((N,), f32),
            pltpu.VMEM((N,), f32),
            pltpu.VMEM((N,), f32),
            pltpu.VMEM((N,), f32),
            pltpu.VMEM((N,), jnp.int32),
            pltpu.VMEM((NB * L,), jnp.int32),
            pltpu.VMEM((CAP + 32,), jnp.int32),
            pltpu.VMEM((CAP + 32,), jnp.int32),
            pltpu.VMEM((NP * 3,), f32),
            pltpu.VMEM((NP * PS * 3,), f32),
        ),
    )
    pf, cf = run(pts)
    return (pf.reshape(B, NP, PS, 3), cf.reshape(B, NP, 3))


# speedup vs baseline: 14.3202x; 1.0031x over previous
import functools

import jax
import jax.numpy as jnp
from jax import lax
from jax.experimental import pallas as pl
from jax.experimental.pallas import tpu as pltpu
from jax.experimental.pallas import tpu_sc as plsc

B = 32
N = 16384
NP = 64
PS = 32
L = 16
NCHUNK = N // L
SHIFT = 22
NB = 512
CAP = 1024
IMAX = 2**31 - 1


def _splat_i(s):
    return jnp.full((L,), s, dtype=jnp.int32)


def _splat_f(s):
    return jnp.full((L,), s, dtype=jnp.float32)


def _rbf16(v):
    u = plsc.bitcast(v, jnp.int32)
    r = (u + 32767 + (lax.shift_right_logical(u, 16) & 1)) & jnp.int32(-65536)
    return plsc.bitcast(r, jnp.float32)


def _sc_body(pts, patches_o, centers_o, xv, yv, zv, x2v, dv, ukv, histv,
             candk, candi, cbuf, pbuf):
    lanes = lax.iota(jnp.int32, L)
    lane0 = lanes == 0
    wid = lax.axis_index("s") * 2 + lax.axis_index("c")
    base = wid * (3 * N)

    pltpu.sync_copy(pts.at[pl.ds(base, N)], xv)
    pltpu.sync_copy(pts.at[pl.ds(base + N, N)], yv)
    pltpu.sync_copy(pts.at[pl.ds(base + 2 * N, N)], zv)

    def x2_body(i, _):
        for u in range(8):
            s = pl.ds((i * 8 + u) * L, L)
            x, y, z = xv[s], yv[s], zv[s]
            x2v[s] = (x * x + y * y) + z * z
            dv[s] = _splat_f(jnp.inf)
        return 0

    lax.fori_loop(0, NCHUNK // 8, x2_body, 0)

    def fps_step(t, cur):
        ci = _splat_i(cur)
        px = plsc.load_gather(xv, [ci])
        py = plsc.load_gather(yv, [ci])
        pz = plsc.load_gather(zv, [ci])
        plsc.store_scatter(cbuf, [_splat_i(t * 3)], px, mask=lane0)
        plsc.store_scatter(cbuf, [_splat_i(t * 3 + 1)], py, mask=lane0)
        plsc.store_scatter(cbuf, [_splat_i(t * 3 + 2)], pz, mask=lane0)

        def chunk(i, carry):
            ms, idxs = list(carry[0]), list(carry[1])
            U = 8
            ss = [pl.ds((i * 8 + u) * L, L) for u in range(U)]
            dx = [xv[s] - px for s in ss]
            dy = [yv[s] - py for s in ss]
            dz = [zv[s] - pz for s in ss]
            od = [dv[s] for s in ss]
            sq = [(dx[u] * dx[u] + dy[u] * dy[u]) for u in range(U)]
            d = [sq[u] + dz[u] * dz[u] for u in range(U)]
            nd = [jnp.minimum(od[u], d[u]) for u in range(U)]
            for u in range(U):
                dv[ss[u]] = nd[u]
            upd = [nd[u] > ms[u % 4] for u in range(4)]
            for u in range(4):
                ms[u] = jnp.where(upd[u], nd[u], ms[u])
                idxs[u] = jnp.where(upd[u], _splat_i((i * 8 + u) * L) + lanes,
                                    idxs[u])
            upd2 = [nd[4 + u] > ms[u] for u in range(4)]
            for u in range(4):
                ms[u] = jnp.where(upd2[u], nd[4 + u], ms[u])
                idxs[u] = jnp.where(upd2[u],
                                    _splat_i((i * 8 + 4 + u) * L) + lanes,
                                    idxs[u])
            return tuple(ms), tuple(idxs)

        ninf = _splat_f(-jnp.inf)
        zi = _splat_i(0)
        ms, idxs = lax.fori_loop(0, NCHUNK // 8, chunk,
                                 ((ninf,) * 4, (zi,) * 4))
        m, idx = ms[0], idxs[0]
        for k in range(1, 4):
            better = jnp.logical_or(
                ms[k] > m, jnp.logical_and(ms[k] == m, idxs[k] < idx))
            m = jnp.where(better, ms[k], m)
            idx = jnp.where(better, idxs[k], idx)
        gm = jnp.max(m)
        return jnp.min(jnp.where(m == gm, idx, IMAX))

    lax.fori_loop(0, NP, fps_step, jnp.int32(0))

    def pack_body(i, _):
        for u in range(8):
            s = pl.ds((i * 8 + u) * L, L)
            ux = plsc.bitcast(_rbf16(xv[s]), jnp.int32)
            uy = plsc.bitcast(_rbf16(yv[s]), jnp.int32)
            dv[s] = plsc.bitcast(
                ux | lax.shift_right_logical(uy, 16), jnp.float32)
        return 0

    lax.fori_loop(0, NCHUNK // 8, pack_body, 0)

    def center_step(p, _):
        cx = plsc.load_gather(cbuf, [_splat_i(p * 3)])
        cy = plsc.load_gather(cbuf, [_splat_i(p * 3 + 1)])
        cz = plsc.load_gather(cbuf, [_splat_i(p * 3 + 2)])
        c2 = (cx * cx + cy * cy) + cz * cz
        cxb, cyb, czb = _rbf16(cx), _rbf16(cy), _rbf16(cz)

        def hzero(i, _):
            for u in range(8):
                histv[pl.ds((i * 8 + u) * L, L)] = _splat_i(0)
            return 0

        lax.fori_loop(0, NB // 8, hzero, 0)

        one = _splat_i(1)
        zf = _splat_f(0.0)
        nbm1 = _splat_i(NB - 1)

        def pass1(i, mn):
            U = 8
            ss = [pl.ds((i * 8 + u) * L, L) for u in range(U)]
            pk = [plsc.bitcast(dv[s], jnp.int32) for s in ss]
            zr = [zv[s] for s in ss]
            xx = [x2v[s] for s in ss]
            xb = [plsc.bitcast(p_ & jnp.int32(-65536), jnp.float32)
                  for p_ in pk]
            yb = [plsc.bitcast(lax.shift_left(p_, 16), jnp.float32)
                  for p_ in pk]
            zb = [_rbf16(z) for z in zr]
            dt = [cxb * xb[u] + cyb * yb[u] for u in range(U)]
            dot = [dt[u] + czb * zb[u] for u in range(U)]
            d2 = [(c2 + xx[u]) - (dot[u] + dot[u]) for u in range(U)]
            ub = [plsc.bitcast(v, jnp.int32) for v in d2]
            uk = [ub[u] ^ lax.shift_right_logical(
                lax.shift_right_arithmetic(ub[u], 31), 1) for u in range(U)]
            for u in range(U):
                ukv[ss[u]] = uk[u]
            bkt = [jnp.minimum(
                lax.shift_right_arithmetic(
                    plsc.bitcast(jnp.maximum(d2[u], zf), jnp.int32),
                    SHIFT), nbm1) for u in range(U)]
            for u in range(U):
                plsc.addupdate_scatter(
                    histv, [lax.shift_left(bkt[u], 4) + lanes], one)
            m1 = [jnp.minimum(d2[2 * u], d2[2 * u + 1]) for u in range(4)]
            m2 = [jnp.minimum(m1[2 * u], m1[2 * u + 1]) for u in range(2)]
            return jnp.minimum(mn, jnp.minimum(m2[0], m2[1]))

        mn = lax.fori_loop(0, NCHUNK // 8, pass1, _splat_f(jnp.inf))
        b0 = jnp.min(jnp.minimum(
            lax.shift_right_arithmetic(
                plsc.bitcast(jnp.maximum(mn, zf), jnp.int32), SHIFT),
            nbm1))

        def tcond(carry):
            bkt, cum = carry
            return jnp.logical_and(cum < PS, bkt < NB)

        def tbody(carry):
            bkt, cum = carry
            row = histv[pl.ds(bkt * L, L)]
            return bkt + 1, cum + jnp.sum(row)

        tend, _ = lax.while_loop(tcond, tbody, (b0, jnp.int32(0)))
        thr = _splat_i(lax.shift_left(tend, SHIFT))

        def collect(i, off):
            U = 8
            ss = [pl.ds((i * 8 + u) * L, L) for u in range(U)]
            uks = [ukv[s] for s in ss]
            msks = [uk < thr for uk in uks]
            csum = [plsc.cumsum(jnp.where(m_, 1, 0)) for m_ in msks]
            pcs = [plsc.all_reduce_population_count(m_) for m_ in msks]
            offs = [off]
            for u in range(U):
                offs.append(offs[u] + pcs[u])
            capv = _splat_i(CAP + 15)
            addr = [jnp.minimum(offs[u] + (csum[u] - 1), capv)
                    for u in range(U)]
            for u in range(U):
                plsc.store_scatter(candk, [addr[u]], uks[u], mask=msks[u])
                plsc.store_scatter(candi, [addr[u]],
                                   _splat_i((i * 8 + u) * L) + lanes,
                                   mask=msks[u])
            return offs[U]

        off = lax.fori_loop(0, NCHUNK // 8, collect, _splat_i(0))
        cnum = jnp.minimum(jnp.max(off), jnp.int32(CAP))
        for q in range(4):
            candk[pl.ds(cnum + q * L, L)] = _splat_i(IMAX)
        nv4 = lax.shift_right_arithmetic(cnum + 63, 6)

        def sel_step(t, carry):
            gk, gi = carry
            gkv, giv = _splat_i(gk), _splat_i(gi)

            def scan(v, c2_):
                bk, bi = c2_
                ss2 = [pl.ds((v * 4 + q) * L, L) for q in range(4)]
                ks = [candk[s] for s in ss2]
                i2s = [candi[s] for s in ss2]
                hits = [jnp.logical_and(ks[q] == gkv, i2s[q] == giv)
                        for q in range(4)]
                ks = [jnp.where(hits[q], IMAX, ks[q]) for q in range(4)]
                for q in range(4):
                    candk[ss2[q]] = ks[q]

                def mrg(ka, ia, kb, ib):
                    lt = jnp.logical_or(
                        kb < ka, jnp.logical_and(kb == ka, ib < ia))
                    return jnp.where(lt, kb, ka), jnp.where(lt, ib, ia)

                k01, i01 = mrg(ks[0], i2s[0], ks[1], i2s[1])
                k23, i23 = mrg(ks[2], i2s[2], ks[3], i2s[3])
                kq, iq = mrg(k01, i01, k23, i23)
                return mrg(bk, bi, kq, iq)

            bk, bi = lax.fori_loop(0, nv4, scan, (_splat_i(IMAX),
                                                  _splat_i(IMAX)))
            ngk = jnp.min(bk)
            ngi = jnp.min(jnp.where(bk == ngk, bi, IMAX))
            giv = _splat_i(jnp.minimum(ngi, jnp.int32(N - 1)))
            pa = _splat_i(p * (PS * 3) + t * 3)
            plsc.store_scatter(pbuf, [pa],
                               plsc.load_gather(xv, [giv]) - cx, mask=lane0)
            plsc.store_scatter(pbuf, [pa + 1],
                               plsc.load_gather(yv, [giv]) - cy, mask=lane0)
            plsc.store_scatter(pbuf, [pa + 2],
                               plsc.load_gather(zv, [giv]) - cz, mask=lane0)
            return ngk, ngi

        lax.fori_loop(0, PS, sel_step, (jnp.int32(IMAX), jnp.int32(-1)))
        return 0

    lax.fori_loop(0, NP, center_step, 0)

    pltpu.sync_copy(pbuf, patches_o.at[pl.ds(wid * (NP * PS * 3),
                                             NP * PS * 3)])
    pltpu.sync_copy(cbuf, centers_o.at[pl.ds(wid * (NP * 3), NP * 3)])


@jax.jit
def kernel(points):
    pts = jnp.transpose(points, (0, 2, 1)).reshape(-1)
    f32 = jnp.float32
    run = pl.kernel(
        _sc_body,
        out_type=(
            jax.ShapeDtypeStruct((B * NP * PS * 3,), f32),
            jax.ShapeDtypeStruct((B * NP * 3,), f32),
        ),
        mesh=plsc.VectorSubcoreMesh(core_axis_name="c", subcore_axis_name="s"),
        compiler_params=pltpu.CompilerParams(
            use_tc_tiling_on_sc=False, needs_layout_passes=False),
        scratch_types=(
            pltpu.VMEM((N,), f32),
            pltpu.VMEM((N,), f32),
            pltpu.VMEM((N,), f32),
            pltpu.VMEM((N,), f32),
            pltpu.VMEM((N,), f32),
            pltpu.VMEM((N,), jnp.int32),
            pltpu.VMEM((NB * L,), jnp.int32),
            pltpu.VMEM((CAP + 96,), jnp.int32),
            pltpu.VMEM((CAP + 96,), jnp.int32),
            pltpu.VMEM((NP * 3,), f32),
            pltpu.VMEM((NP * PS * 3,), f32),
        ),
    )
    pf, cf = run(pts)
    return (pf.reshape(B, NP, PS, 3), cf.reshape(B, NP, 3))


# speedup vs baseline: 14.3207x; 1.0000x over previous
import jax
import jax.numpy as jnp
from jax import lax
from jax.experimental import pallas as pl
from jax.experimental.pallas import tpu as pltpu
from jax.experimental.pallas import tpu_sc as plsc

B = 32
N = 16384
NP = 64
PS = 32
L = 16
NCHUNK = N // L
SHIFT = 22
NB = 512
CAP = 1024
IMAX = 2**31 - 1


def _splat_i(s):
    return jnp.full((L,), s, dtype=jnp.int32)


def _splat_f(s):
    return jnp.full((L,), s, dtype=jnp.float32)


def _rbf16(v):
    u = plsc.bitcast(v, jnp.int32)
    r = (u + 32767 + (lax.shift_right_logical(u, 16) & 1)) & jnp.int32(-65536)
    return plsc.bitcast(r, jnp.float32)


def _sc_body(pts, patches_o, centers_o, xv, yv, zv, x2v, dv, ukv, histv,
             candk, candi, cbuf, pbuf):
    lanes = lax.iota(jnp.int32, L)
    lane0 = lanes == 0
    wid = lax.axis_index("s") * 2 + lax.axis_index("c")
    base = wid * (3 * N)

    pltpu.sync_copy(pts.at[pl.ds(base, N)], xv)
    pltpu.sync_copy(pts.at[pl.ds(base + N, N)], yv)
    pltpu.sync_copy(pts.at[pl.ds(base + 2 * N, N)], zv)

    def x2_body(i, _):
        for u in range(8):
            s = pl.ds((i * 8 + u) * L, L)
            x, y, z = xv[s], yv[s], zv[s]
            x2v[s] = (x * x + y * y) + z * z
            dv[s] = _splat_f(jnp.inf)
        return 0

    lax.fori_loop(0, NCHUNK // 8, x2_body, 0)

    def fps_step(t, cur):
        ci = _splat_i(cur)
        px = plsc.load_gather(xv, [ci])
        py = plsc.load_gather(yv, [ci])
        pz = plsc.load_gather(zv, [ci])
        plsc.store_scatter(cbuf, [_splat_i(t * 3)], px, mask=lane0)
        plsc.store_scatter(cbuf, [_splat_i(t * 3 + 1)], py, mask=lane0)
        plsc.store_scatter(cbuf, [_splat_i(t * 3 + 2)], pz, mask=lane0)

        def chunk(i, carry):
            ms, idxs = list(carry[0]), list(carry[1])
            U = 8
            ss = [pl.ds((i * 8 + u) * L, L) for u in range(U)]
            dx = [xv[s] - px for s in ss]
            dy = [yv[s] - py for s in ss]
            dz = [zv[s] - pz for s in ss]
            od = [dv[s] for s in ss]
            sq = [(dx[u] * dx[u] + dy[u] * dy[u]) for u in range(U)]
            d = [sq[u] + dz[u] * dz[u] for u in range(U)]
            nd = [jnp.minimum(od[u], d[u]) for u in range(U)]
            for u in range(U):
                dv[ss[u]] = nd[u]
            upd = [nd[u] > ms[u % 4] for u in range(4)]
            for u in range(4):
                ms[u] = jnp.where(upd[u], nd[u], ms[u])
                idxs[u] = jnp.where(upd[u], _splat_i((i * 8 + u) * L) + lanes,
                                    idxs[u])
            upd2 = [nd[4 + u] > ms[u] for u in range(4)]
            for u in range(4):
                ms[u] = jnp.where(upd2[u], nd[4 + u], ms[u])
                idxs[u] = jnp.where(upd2[u],
                                    _splat_i((i * 8 + 4 + u) * L) + lanes,
                                    idxs[u])
            return tuple(ms), tuple(idxs)

        ninf = _splat_f(-jnp.inf)
        zi = _splat_i(0)
        ms, idxs = lax.fori_loop(0, NCHUNK // 8, chunk,
                                 ((ninf,) * 4, (zi,) * 4))
        m, idx = ms[0], idxs[0]
        for k in range(1, 4):
            better = jnp.logical_or(
                ms[k] > m, jnp.logical_and(ms[k] == m, idxs[k] < idx))
            m = jnp.where(better, ms[k], m)
            idx = jnp.where(better, idxs[k], idx)
        gm = jnp.max(m)
        return jnp.min(jnp.where(m == gm, idx, IMAX))

    lax.fori_loop(0, NP, fps_step, jnp.int32(0))

    def pack_body(i, _):
        for u in range(8):
            s = pl.ds((i * 8 + u) * L, L)
            ux = plsc.bitcast(_rbf16(xv[s]), jnp.int32)
            uy = plsc.bitcast(_rbf16(yv[s]), jnp.int32)
            dv[s] = plsc.bitcast(
                ux | lax.shift_right_logical(uy, 16), jnp.float32)
        return 0

    lax.fori_loop(0, NCHUNK // 8, pack_body, 0)

    def center_step(p, _):
        cx = plsc.load_gather(cbuf, [_splat_i(p * 3)])
        cy = plsc.load_gather(cbuf, [_splat_i(p * 3 + 1)])
        cz = plsc.load_gather(cbuf, [_splat_i(p * 3 + 2)])
        c2 = (cx * cx + cy * cy) + cz * cz
        cxb, cyb, czb = _rbf16(cx), _rbf16(cy), _rbf16(cz)

        def hzero(i, _):
            for u in range(8):
                histv[pl.ds((i * 8 + u) * L, L)] = _splat_i(0)
            return 0

        lax.fori_loop(0, NB // 8, hzero, 0)

        one = _splat_i(1)
        zf = _splat_f(0.0)
        nbm1 = _splat_i(NB - 1)

        def pass1(i, mn):
            U = 8
            ss = [pl.ds((i * 8 + u) * L, L) for u in range(U)]
            pk = [plsc.bitcast(dv[s], jnp.int32) for s in ss]
            zr = [zv[s] for s in ss]
            xx = [x2v[s] for s in ss]
            xb = [plsc.bitcast(p_ & jnp.int32(-65536), jnp.float32)
                  for p_ in pk]
            yb = [plsc.bitcast(lax.shift_left(p_, 16), jnp.float32)
                  for p_ in pk]
            zb = [_rbf16(z) for z in zr]
            dt = [cxb * xb[u] + cyb * yb[u] for u in range(U)]
            dot = [dt[u] + czb * zb[u] for u in range(U)]
            d2 = [(c2 + xx[u]) - (dot[u] + dot[u]) for u in range(U)]
            ub = [plsc.bitcast(v, jnp.int32) for v in d2]
            uk = [ub[u] ^ lax.shift_right_logical(
                lax.shift_right_arithmetic(ub[u], 31), 1) for u in range(U)]
            for u in range(U):
                ukv[ss[u]] = uk[u]
            bkt = [jnp.minimum(
                lax.shift_right_arithmetic(
                    plsc.bitcast(jnp.maximum(d2[u], zf), jnp.int32),
                    SHIFT), nbm1) for u in range(U)]
            for u in range(U):
                plsc.addupdate_scatter(
                    histv, [lax.shift_left(bkt[u], 4) + lanes], one)
            m1 = [jnp.minimum(d2[2 * u], d2[2 * u + 1]) for u in range(4)]
            m2 = [jnp.minimum(m1[2 * u], m1[2 * u + 1]) for u in range(2)]
            return jnp.minimum(mn, jnp.minimum(m2[0], m2[1]))

        mn = lax.fori_loop(0, NCHUNK // 8, pass1, _splat_f(jnp.inf))
        b0 = jnp.min(jnp.minimum(
            lax.shift_right_arithmetic(
                plsc.bitcast(jnp.maximum(mn, zf), jnp.int32), SHIFT),
            nbm1))

        def tcond(carry):
            bkt, cum = carry
            return jnp.logical_and(cum < PS, bkt < NB)

        def tbody(carry):
            bkt, cum = carry
            row = histv[pl.ds(bkt * L, L)]
            return bkt + 1, cum + jnp.sum(row)

        tend, _ = lax.while_loop(tcond, tbody, (b0, jnp.int32(0)))
        thr = _splat_i(lax.shift_left(tend, SHIFT))

        def collect(i, off):
            U = 8
            ss = [pl.ds((i * 8 + u) * L, L) for u in range(U)]
            uks = [ukv[s] for s in ss]
            msks = [uk < thr for uk in uks]
            csum = [plsc.cumsum(jnp.where(m_, 1, 0)) for m_ in msks]
            pcs = [plsc.all_reduce_population_count(m_) for m_ in msks]
            offs = [off]
            for u in range(U):
                offs.append(offs[u] + pcs[u])
            capv = _splat_i(CAP + 15)
            addr = [jnp.minimum(offs[u] + (csum[u] - 1), capv)
                    for u in range(U)]
            for u in range(U):
                plsc.store_scatter(candk, [addr[u]], uks[u], mask=msks[u])
                plsc.store_scatter(candi, [addr[u]],
                                   _splat_i((i * 8 + u) * L) + lanes,
                                   mask=msks[u])
            return offs[U]

        off = lax.fori_loop(0, NCHUNK // 8, collect, _splat_i(0))
        cnum = jnp.minimum(jnp.max(off), jnp.int32(CAP))
        for q in range(4):
            candk[pl.ds(cnum + q * L, L)] = _splat_i(IMAX)
        nv4 = lax.shift_right_arithmetic(cnum + 63, 6)

        def sel_step(t, carry):
            gk, gi = carry
            gkv, giv = _splat_i(gk), _splat_i(gi)

            def scan(v, c2_):
                bk, bi = c2_
                ss2 = [pl.ds((v * 4 + q) * L, L) for q in range(4)]
                ks = [candk[s] for s in ss2]
                i2s = [candi[s] for s in ss2]
                hits = [jnp.logical_and(ks[q] == gkv, i2s[q] == giv)
                        for q in range(4)]
                ks = [jnp.where(hits[q], IMAX, ks[q]) for q in range(4)]
                for q in range(4):
                    candk[ss2[q]] = ks[q]

                def mrg(ka, ia, kb, ib):
                    lt = jnp.logical_or(
                        kb < ka, jnp.logical_and(kb == ka, ib < ia))
                    return jnp.where(lt, kb, ka), jnp.where(lt, ib, ia)

                k01, i01 = mrg(ks[0], i2s[0], ks[1], i2s[1])
                k23, i23 = mrg(ks[2], i2s[2], ks[3], i2s[3])
                kq, iq = mrg(k01, i01, k23, i23)
                return mrg(bk, bi, kq, iq)

            bk, bi = lax.fori_loop(0, nv4, scan, (_splat_i(IMAX),
                                                  _splat_i(IMAX)))
            ngk = jnp.min(bk)
            ngi = jnp.min(jnp.where(bk == ngk, bi, IMAX))
            giv = _splat_i(jnp.minimum(ngi, jnp.int32(N - 1)))
            pa = _splat_i(p * (PS * 3) + t * 3)
            plsc.store_scatter(pbuf, [pa],
                               plsc.load_gather(xv, [giv]) - cx, mask=lane0)
            plsc.store_scatter(pbuf, [pa + 1],
                               plsc.load_gather(yv, [giv]) - cy, mask=lane0)
            plsc.store_scatter(pbuf, [pa + 2],
                               plsc.load_gather(zv, [giv]) - cz, mask=lane0)
            return ngk, ngi

        lax.fori_loop(0, PS, sel_step, (jnp.int32(IMAX), jnp.int32(-1)))
        return 0

    lax.fori_loop(0, NP, center_step, 0)

    pltpu.sync_copy(pbuf, patches_o.at[pl.ds(wid * (NP * PS * 3),
                                             NP * PS * 3)])
    pltpu.sync_copy(cbuf, centers_o.at[pl.ds(wid * (NP * 3), NP * 3)])


@jax.jit
def kernel(points):
    pts = jnp.transpose(points, (0, 2, 1)).reshape(-1)
    f32 = jnp.float32
    run = pl.kernel(
        _sc_body,
        out_type=(
            jax.ShapeDtypeStruct((B * NP * PS * 3,), f32),
            jax.ShapeDtypeStruct((B * NP * 3,), f32),
        ),
        mesh=plsc.VectorSubcoreMesh(core_axis_name="c", subcore_axis_name="s"),
        compiler_params=pltpu.CompilerParams(
            use_tc_tiling_on_sc=False, needs_layout_passes=False),
        scratch_types=(
            pltpu.VMEM((N,), f32),
            pltpu.VMEM((N,), f32),
            pltpu.VMEM((N,), f32),
            pltpu.VMEM((N,), f32),
            pltpu.VMEM((N,), f32),
            pltpu.VMEM((N,), jnp.int32),
            pltpu.VMEM((NB * L,), jnp.int32),
            pltpu.VMEM((CAP + 96,), jnp.int32),
            pltpu.VMEM((CAP + 96,), jnp.int32),
            pltpu.VMEM((NP * 3,), f32),
            pltpu.VMEM((NP * PS * 3,), f32),
        ),
    )
    pf, cf = run(pts)
    return (pf.reshape(B, NP, PS, 3), cf.reshape(B, NP, 3))
